# baseline (speedup 1.0000x reference)
import functools

import jax
import jax.numpy as jnp
from jax import lax
from jax.experimental import pallas as pl
from jax.experimental.pallas import tpu as pltpu
from jax.experimental.pallas import tpu_sc as plsc

_NC = 2
_NS = 16
_NW = _NC * _NS
_L = 16
_TR = 8


@functools.lru_cache(maxsize=None)
def _build(n, feat, emb, dtype_name):
    dtype = jnp.dtype(dtype_name)
    out_w = feat + emb
    assert out_w % _TR == 0
    rows_per_w = n // _NW
    chunk = min(128, rows_per_w)
    nchunk = rows_per_w // chunk
    nbuf = 2
    mesh = plsc.VectorSubcoreMesh(core_axis_name="c", subcore_axis_name="s")

    @functools.partial(
        pl.kernel,
        mesh=mesh,
        compiler_params=pltpu.CompilerParams(needs_layout_passes=False),
        out_type=jax.ShapeDtypeStruct((out_w, n), dtype),
        scratch_types=[
            *[pltpu.VMEM((chunk,), jnp.int32) for _ in range(nbuf)],
            *[pltpu.VMEM((chunk,), jnp.int32) for _ in range(nbuf)],
            *[pltpu.VMEM((feat, chunk), dtype) for _ in range(nbuf)],
            *[pltpu.VMEM((chunk, emb), dtype) for _ in range(nbuf)],
            *[pltpu.VMEM((chunk, emb), dtype) for _ in range(nbuf)],
            *[pltpu.VMEM((out_w, chunk), dtype) for _ in range(nbuf)],
            *[pltpu.SemaphoreType.DMA for _ in range(nbuf)],
            *[pltpu.SemaphoreType.DMA for _ in range(nbuf)],
            *[pltpu.SemaphoreType.DMA for _ in range(nbuf)],
            *[pltpu.SemaphoreType.DMA for _ in range(nbuf)],
        ],
    )
    def sc_kernel(lanesT_hbm, rid_hbm, lid_hbm, rtab_hbm, ltab_hbm, outT_hbm,
                  *scr):
        rid_v = scr[0:nbuf]
        lid_v = scr[nbuf:2 * nbuf]
        feat_v = scr[2 * nbuf:3 * nbuf]
        rrow_v = scr[3 * nbuf:4 * nbuf]
        lrow_v = scr[4 * nbuf:5 * nbuf]
        stg_v = scr[5 * nbuf:6 * nbuf]
        sem_r = scr[6 * nbuf:7 * nbuf]
        sem_l = scr[7 * nbuf:8 * nbuf]
        sem_o = scr[8 * nbuf:9 * nbuf]
        sem_f = scr[9 * nbuf:10 * nbuf]

        wid = lax.axis_index("s") * _NC + lax.axis_index("c")
        base = wid * rows_per_w

        iota = lax.iota(jnp.int32, _L)
        zero = iota * 0

        def start_fetch(c):
            b = c % nbuf
            lane0 = base + c * chunk
            pltpu.sync_copy(rid_hbm.at[pl.ds(lane0, chunk)], rid_v[b])
            pltpu.sync_copy(lid_hbm.at[pl.ds(lane0, chunk)], lid_v[b])
            cp_r = pltpu.async_copy(rtab_hbm.at[rid_v[b]], rrow_v[b], sem_r[b])
            cp_l = pltpu.async_copy(ltab_hbm.at[lid_v[b]], lrow_v[b], sem_l[b])
            cp_f = pltpu.async_copy(lanesT_hbm.at[:, pl.ds(lane0, chunk)],
                                    feat_v[b], sem_f[b])
            return cp_r, cp_l, cp_f

        fetches = {0: start_fetch(0)}
        out_cps = {}
        for c in range(nchunk):
            b = c % nbuf
            lane0 = base + c * chunk
            if c + 1 < nchunk:
                fetches[c + 1] = start_fetch(c + 1)
            if c - nbuf >= 0:
                for cp in out_cps.pop(c - nbuf):
                    cp.wait()
            cp_r, cp_l, cp_f = fetches.pop(c)
            cp_f.wait()
            cp_r.wait()
            cp_l.wait()

            @plsc.parallel_loop(0, chunk // _L)
            def copy_feats(g, _b=b):
                j0 = g * _L
                for f in range(feat):
                    stg_v[_b][f, pl.ds(j0, _L)] = feat_v[_b][f, pl.ds(j0, _L)]

            lanevecs = [iota + g0 * _L for g0 in range(chunk // _L)]

            @plsc.parallel_loop(0, emb, carry=zero)
            def add_cols(e, ev, _b=b):
                row = feat + e
                vals = []
                for lv in lanevecs:
                    vals.append(plsc.load_gather(rrow_v[_b], [lv, ev])
                                + plsc.load_gather(lrow_v[_b], [lv, ev]))
                for g0, v in enumerate(vals):
                    stg_v[_b][row, pl.ds(g0 * _L, _L)] = v
                return ev + 1

            cps = []
            for r in range(out_w // _TR):
                cps.append(pltpu.async_copy(
                    stg_v[b].at[pl.ds(r * _TR, _TR)],
                    outT_hbm.at[pl.ds(r * _TR, _TR), pl.ds(lane0, chunk)],
                    sem_o[b]))
            out_cps[c] = cps
        for c in sorted(out_cps):
            for cp in out_cps.pop(c):
                cp.wait()

    return sc_kernel


def kernel(lanes, road_id, lane_id, road_table, lane_table):
    n, feat = lanes.shape
    emb = road_table.shape[1]
    fn = _build(n, feat, emb, str(road_table.dtype))
    outT = fn(lanes.T,
              road_id.astype(jnp.int32),
              lane_id.astype(jnp.int32),
              road_table,
              lane_table)
    return outT.T


# speedup vs baseline: 1.8723x; 1.8723x over previous
import functools

import jax
import jax.numpy as jnp
from jax import lax
from jax.experimental import pallas as pl
from jax.experimental.pallas import tpu as pltpu
from jax.experimental.pallas import tpu_sc as plsc

_NC = 2
_NS = 16
_NW = _NC * _NS
_L = 16


@functools.lru_cache(maxsize=None)
def _build(n, feat, emb, dtype_name):
    dtype = jnp.dtype(dtype_name)
    out_w = feat + emb
    rows_per_w = n // _NW
    chunk = min(64, rows_per_w)
    nchunk = rows_per_w // chunk
    nbuf = 2
    mesh = plsc.VectorSubcoreMesh(core_axis_name="c", subcore_axis_name="s")

    n_slice = emb // _L - 1
    tail_src = n_slice * _L

    @functools.partial(
        pl.kernel,
        mesh=mesh,
        compiler_params=pltpu.CompilerParams(needs_layout_passes=False),
        out_type=jax.ShapeDtypeStruct((n, out_w), dtype),
        scratch_types=[
            *[pltpu.VMEM((chunk,), jnp.int32) for _ in range(nbuf)],
            *[pltpu.VMEM((chunk,), jnp.int32) for _ in range(nbuf)],
            *[pltpu.VMEM((feat, 2 * chunk), dtype) for _ in range(nbuf)],
            *[pltpu.VMEM((chunk, emb), dtype) for _ in range(nbuf)],
            *[pltpu.VMEM((chunk, emb), dtype) for _ in range(nbuf)],
            *[pltpu.VMEM((chunk, out_w), dtype) for _ in range(nbuf)],
            *[pltpu.SemaphoreType.DMA for _ in range(nbuf)],
            *[pltpu.SemaphoreType.DMA for _ in range(nbuf)],
            *[pltpu.SemaphoreType.DMA for _ in range(nbuf)],
            *[pltpu.SemaphoreType.DMA for _ in range(nbuf)],
        ],
    )
    def sc_kernel(lanesT_hbm, rid_hbm, lid_hbm, rtab_hbm, ltab_hbm, out_hbm,
                  *scr):
        rid_v = scr[0:nbuf]
        lid_v = scr[nbuf:2 * nbuf]
        feat_v = scr[2 * nbuf:3 * nbuf]
        rrow_v = scr[3 * nbuf:4 * nbuf]
        lrow_v = scr[4 * nbuf:5 * nbuf]
        out_v = scr[5 * nbuf:6 * nbuf]
        sem_r = scr[6 * nbuf:7 * nbuf]
        sem_l = scr[7 * nbuf:8 * nbuf]
        sem_o = scr[8 * nbuf:9 * nbuf]
        sem_f = scr[9 * nbuf:10 * nbuf]

        wid = lax.axis_index("s") * _NC + lax.axis_index("c")
        base = wid * rows_per_w

        iota = lax.iota(jnp.int32, _L)
        rvec0 = jnp.where(iota >= feat, 1, 0)
        for k in range(2, _L // feat):
            rvec0 = rvec0 + jnp.where(iota >= k * feat, 1, 0)
        cvec_f = iota - rvec0 * feat
        rstep = _L // feat
        cvec_t = iota + (out_w - _L)

        def start_fetch(c):
            b = c % nbuf
            row0 = base + c * chunk
            pltpu.sync_copy(rid_hbm.at[pl.ds(row0, chunk)], rid_v[b])
            pltpu.sync_copy(lid_hbm.at[pl.ds(row0, chunk)], lid_v[b])
            cp_r = pltpu.async_copy(rtab_hbm.at[rid_v[b]], rrow_v[b], sem_r[b])
            cp_l = pltpu.async_copy(ltab_hbm.at[lid_v[b]], lrow_v[b], sem_l[b])
            cp_f = None
            if c % 2 == 0:
                cp_f = pltpu.async_copy(
                    lanesT_hbm.at[:, pl.ds(row0, 2 * chunk)],
                    feat_v[(c // 2) % nbuf], sem_f[(c // 2) % nbuf])
            return cp_r, cp_l, cp_f

        fetches = {0: start_fetch(0)}
        out_cps = {}
        for c in range(nchunk):
            b = c % nbuf
            row0 = base + c * chunk
            if c + 1 < nchunk:
                fetches[c + 1] = start_fetch(c + 1)
            if c - nbuf >= 0:
                out_cps.pop(c - nbuf).wait()
            cp_r, cp_l, cp_f = fetches.pop(c)
            if cp_f is not None:
                cp_f.wait()
            fb, fo = (c // 2) % nbuf, (c % 2) * chunk
            for f in range(feat):
                for g in range(chunk // _L):
                    lv = iota + g * _L
                    vals = feat_v[fb][f, pl.ds(fo + g * _L, _L)]
                    plsc.store_scatter(out_v[b], [lv, lv * 0 + f], vals)
            cp_r.wait()
            cp_l.wait()

            @plsc.parallel_loop(0, chunk, carry=iota * 0)
            def add_row(rr, rv, _b=b):
                for j in range(n_slice):
                    out_v[_b][rr, pl.ds(feat + j * _L, _L)] = (
                        rrow_v[_b][rr, pl.ds(j * _L, _L)]
                        + lrow_v[_b][rr, pl.ds(j * _L, _L)])
                tail = (rrow_v[_b][rr, pl.ds(tail_src, _L)]
                        + lrow_v[_b][rr, pl.ds(tail_src, _L)])
                plsc.store_scatter(out_v[_b], [rv, cvec_t], tail)
                return rv + 1

            out_cps[c] = pltpu.async_copy(
                out_v[b], out_hbm.at[pl.ds(row0, chunk)], sem_o[b])
        for c in sorted(out_cps):
            out_cps.pop(c).wait()

    return sc_kernel


def kernel(lanes, road_id, lane_id, road_table, lane_table):
    n, feat = lanes.shape
    emb = road_table.shape[1]
    fn = _build(n, feat, emb, str(road_table.dtype))
    return fn(lanes.T,
              road_id.astype(jnp.int32),
              lane_id.astype(jnp.int32),
              road_table,
              lane_table)


# speedup vs baseline: 1.9160x; 1.0233x over previous
import functools

import jax
import jax.numpy as jnp
from jax import lax
from jax.experimental import pallas as pl
from jax.experimental.pallas import tpu as pltpu
from jax.experimental.pallas import tpu_sc as plsc

_NC = 2
_NS = 16
_NW = _NC * _NS
_L = 16


@functools.lru_cache(maxsize=None)
def _build(n, feat, emb, dtype_name):
    dtype = jnp.dtype(dtype_name)
    out_w = feat + emb
    rows_per_w = n // _NW
    chunk = min(128, rows_per_w)
    half = chunk // 2
    nchunk = rows_per_w // chunk
    nbuf = 2
    mesh = plsc.VectorSubcoreMesh(core_axis_name="c", subcore_axis_name="s")

    n_slice = emb // _L - 1
    tail_src = n_slice * _L

    @functools.partial(
        pl.kernel,
        mesh=mesh,
        compiler_params=pltpu.CompilerParams(needs_layout_passes=False),
        out_type=jax.ShapeDtypeStruct((n, out_w), dtype),
        scratch_types=[
            *[pltpu.VMEM((chunk,), jnp.int32) for _ in range(nbuf)],
            *[pltpu.VMEM((chunk,), jnp.int32) for _ in range(nbuf)],
            *[pltpu.VMEM((feat, chunk), dtype) for _ in range(nbuf)],
            *[pltpu.VMEM((chunk, emb), dtype) for _ in range(nbuf)],
            *[pltpu.VMEM((chunk, emb), dtype) for _ in range(nbuf)],
            *[pltpu.VMEM((half, out_w), dtype) for _ in range(nbuf)],
            *[pltpu.SemaphoreType.DMA for _ in range(nbuf)],
            *[pltpu.SemaphoreType.DMA for _ in range(nbuf)],
            *[pltpu.SemaphoreType.DMA for _ in range(nbuf)],
            *[pltpu.SemaphoreType.DMA for _ in range(nbuf)],
        ],
    )
    def sc_kernel(lanesT_hbm, rid_hbm, lid_hbm, rtab_hbm, ltab_hbm, out_hbm,
                  *scr):
        rid_v = scr[0:nbuf]
        lid_v = scr[nbuf:2 * nbuf]
        feat_v = scr[2 * nbuf:3 * nbuf]
        rrow_v = scr[3 * nbuf:4 * nbuf]
        lrow_v = scr[4 * nbuf:5 * nbuf]
        out_v = scr[5 * nbuf:6 * nbuf]
        sem_r = scr[6 * nbuf:7 * nbuf]
        sem_l = scr[7 * nbuf:8 * nbuf]
        sem_o = scr[8 * nbuf:9 * nbuf]
        sem_f = scr[9 * nbuf:10 * nbuf]

        wid = lax.axis_index("s") * _NC + lax.axis_index("c")
        base = wid * rows_per_w

        iota = lax.iota(jnp.int32, _L)
        rvec0 = jnp.where(iota >= feat, 1, 0)
        for k in range(2, _L // feat):
            rvec0 = rvec0 + jnp.where(iota >= k * feat, 1, 0)
        cvec_f = iota - rvec0 * feat
        rstep = _L // feat
        cvec_t = iota + (out_w - _L)

        def start_fetch(c):
            b = c % nbuf
            row0 = base + c * chunk
            pltpu.sync_copy(rid_hbm.at[pl.ds(row0, chunk)], rid_v[b])
            pltpu.sync_copy(lid_hbm.at[pl.ds(row0, chunk)], lid_v[b])
            cp_r = pltpu.async_copy(rtab_hbm.at[rid_v[b]], rrow_v[b], sem_r[b])
            cp_l = pltpu.async_copy(ltab_hbm.at[lid_v[b]], lrow_v[b], sem_l[b])
            cp_f = pltpu.async_copy(lanesT_hbm.at[:, pl.ds(row0, chunk)],
                                    feat_v[b], sem_f[b])
            return cp_r, cp_l, cp_f

        fetches = {0: start_fetch(0)}
        out_cps = {}
        for c in range(nchunk):
            b = c % nbuf
            row0 = base + c * chunk
            if c + 1 < nchunk:
                fetches[c + 1] = start_fetch(c + 1)
            cp_r, cp_l, cp_f = fetches.pop(c)
            cp_f.wait()
            cp_r.wait()
            cp_l.wait()
            for h in range(2):
                hb = h
                hr0 = h * half
                if (c, h) != (0, 0) and (c, h) != (0, 1):
                    out_cps.pop((c - 1, h)).wait()
                for f in range(feat):
                    for g in range(half // _L):
                        lv = iota + g * _L
                        vals = feat_v[b][f, pl.ds(hr0 + g * _L, _L)]
                        plsc.store_scatter(out_v[hb], [lv, lv * 0 + f], vals)

                @plsc.parallel_loop(0, half, carry=iota * 0)
                def add_row(rr, rv, _b=b, _hb=hb, _hr0=hr0):
                    for j in range(n_slice):
                        out_v[_hb][rr, pl.ds(feat + j * _L, _L)] = (
                            rrow_v[_b][_hr0 + rr, pl.ds(j * _L, _L)]
                            + lrow_v[_b][_hr0 + rr, pl.ds(j * _L, _L)])
                    tail = (rrow_v[_b][_hr0 + rr, pl.ds(tail_src, _L)]
                            + lrow_v[_b][_hr0 + rr, pl.ds(tail_src, _L)])
                    plsc.store_scatter(out_v[_hb], [rv, cvec_t], tail)
                    return rv + 1

                out_cps[(c, h)] = pltpu.async_copy(
                    out_v[hb], out_hbm.at[pl.ds(row0 + hr0, half)], sem_o[hb])
        for k in sorted(out_cps):
            out_cps.pop(k).wait()

    return sc_kernel


def kernel(lanes, road_id, lane_id, road_table, lane_table):
    n, feat = lanes.shape
    emb = road_table.shape[1]
    fn = _build(n, feat, emb, str(road_table.dtype))
    return fn(lanes.T,
              road_id.astype(jnp.int32),
              lane_id.astype(jnp.int32),
              road_table,
              lane_table)
